# Initial kernel scaffold; baseline (speedup 1.0000x reference)
import jax
import jax.numpy as jnp
from jax.experimental import pallas as pl


def kernel(x, edge_index, W1, b1, W2, b2, Wc, bc):
    raise NotImplementedError("write your pallas kernel here")


# speedup vs baseline: 4.4389x; 4.4389x over previous
import functools

import jax
import jax.numpy as jnp
from jax import lax
from jax.experimental import pallas as pl
from jax.experimental.pallas import tpu as pltpu
from jax.experimental.pallas import tpu_sc as plsc

NC = 2
NS = 16
LANES = 16
F = 128
BN = 400


def _sc_mesh():
    return plsc.VectorSubcoreMesh(core_axis_name="c", subcore_axis_name="s")


_CHD = 40


def _degree_body(n_pad, epw, dst_hbm, ones_hbm, z_hbm, out_hbm, didx, onesv,
                 acc):
    c = lax.axis_index("c")
    s = lax.axis_index("s")
    wid = c * NS + s
    npw = n_pad // NS

    pltpu.sync_copy(ones_hbm, onesv)
    pltpu.sync_copy(z_hbm.at[pl.ds(s * npw, npw)], acc.at[pl.ds(s * npw, npw)])
    plsc.subcore_barrier()

    base = wid * epw

    def cbody(k, carry):
        pltpu.sync_copy(dst_hbm.at[pl.ds(base + k * _CHD, _CHD)], didx)
        pltpu.sync_copy(onesv, acc.at[didx], add=True)
        return carry

    lax.fori_loop(0, epw // _CHD, cbody, 0)
    plsc.subcore_barrier()
    pltpu.sync_copy(acc.at[pl.ds(s * npw, npw)],
                    out_hbm.at[pl.ds(c * n_pad + s * npw, npw)])


def _sc_degree(dst, ones, zeros2d, n_pad, e):
    epw = e // (NC * NS)
    kfn = pl.kernel(
        functools.partial(_degree_body, n_pad, epw),
        out_type=jax.ShapeDtypeStruct((NC * n_pad, F), jnp.float32),
        mesh=_sc_mesh(),
        scratch_types=[
            pltpu.VMEM((_CHD,), jnp.int32),
            pltpu.VMEM((_CHD, F), jnp.float32),
            pltpu.VMEM_SHARED((n_pad, F), jnp.float32),
        ],
    )
    return kfn(dst, ones, zeros2d)


_CH = 80


def _prop_body(n_pad, e, tlo_hbm, thi_hbm, src_hbm, dst_hbm, z_hbm, olo_hbm,
               ohi_hbm, sidx, didx, rows, acc, sem):
    c = lax.axis_index("c")
    s = lax.axis_index("s")
    ept = e // NS
    nch = ept // _CH
    rpt = n_pad // NS

    pltpu.sync_copy(z_hbm.at[pl.ds(s * rpt, rpt)], acc.at[pl.ds(s * rpt, rpt)])
    plsc.subcore_barrier()
    ebase = s * ept

    def edge_loop(table):
        def cbody(k, carry):
            off = ebase + k * _CH
            pltpu.sync_copy(src_hbm.at[pl.ds(off, _CH)], sidx)
            gather = pltpu.async_copy(table.at[sidx], rows, sem)
            pltpu.sync_copy(dst_hbm.at[pl.ds(off, _CH)], didx)
            gather.wait()
            pltpu.sync_copy(rows, acc.at[didx], add=True)
            return carry

        lax.fori_loop(0, nch, cbody, 0)

    @pl.when(c == 0)
    def _():
        edge_loop(tlo_hbm)

    @pl.when(c == 1)
    def _():
        edge_loop(thi_hbm)

    plsc.subcore_barrier()

    @pl.when(c == 0)
    def _():
        pltpu.sync_copy(acc.at[pl.ds(s * rpt, rpt)],
                        olo_hbm.at[pl.ds(s * rpt, rpt)])

    @pl.when(c == 1)
    def _():
        pltpu.sync_copy(acc.at[pl.ds(s * rpt, rpt)],
                        ohi_hbm.at[pl.ds(s * rpt, rpt)])


def _sc_prop(tlo, thi, src, dst, zeros, n_pad):
    e = src.shape[0]
    kfn = pl.kernel(
        functools.partial(_prop_body, n_pad, e),
        out_type=(jax.ShapeDtypeStruct((n_pad, F), jnp.float32),
                  jax.ShapeDtypeStruct((n_pad, F), jnp.float32)),
        mesh=_sc_mesh(),
        scratch_types=[
            pltpu.VMEM((_CH,), jnp.int32),
            pltpu.VMEM((_CH,), jnp.int32),
            pltpu.VMEM((_CH, F), jnp.float32),
            pltpu.VMEM_SHARED((n_pad, F), jnp.float32),
            pltpu.SemaphoreType.DMA,
        ],
    )
    return kfn(tlo, thi, src, dst, zeros)


def _prep_body(deg_ref, x_ref, t0lo_ref, t0hi_ref, norm_ref, norm2_ref):
    d = jnp.maximum(deg_ref[...], 1.0)
    norm = lax.rsqrt(d)
    xb = x_ref[...]
    t0lo_ref[...] = xb[:, :F] * norm
    t0hi_ref[...] = xb[:, F:] * norm
    norm_ref[...] = norm
    norm2_ref[...] = 1.0 / d


def _tc_prep(deg2, x):
    n = x.shape[0]
    g = n // BN
    return pl.pallas_call(
        _prep_body,
        grid=(g,),
        in_specs=[
            pl.BlockSpec((BN, 1), lambda i: (i, 0)),
            pl.BlockSpec((BN, 2 * F), lambda i: (i, 0)),
        ],
        out_specs=[
            pl.BlockSpec((BN, F), lambda i: (i, 0)),
            pl.BlockSpec((BN, F), lambda i: (i, 0)),
            pl.BlockSpec((BN, 1), lambda i: (i, 0)),
            pl.BlockSpec((BN, 1), lambda i: (i, 0)),
        ],
        out_shape=[
            jax.ShapeDtypeStruct((n, F), jnp.float32),
            jax.ShapeDtypeStruct((n, F), jnp.float32),
            jax.ShapeDtypeStruct((n, 1), jnp.float32),
            jax.ShapeDtypeStruct((n, 1), jnp.float32),
        ],
    )(deg2, x)


def _scale_body(alo_ref, ahi_ref, s_ref, olo_ref, ohi_ref):
    sb = s_ref[...]
    olo_ref[...] = alo_ref[...] * sb
    ohi_ref[...] = ahi_ref[...] * sb


def _tc_scale(alo, ahi, s):
    n = alo.shape[0]
    g = n // BN
    return pl.pallas_call(
        _scale_body,
        grid=(g,),
        in_specs=[
            pl.BlockSpec((BN, F), lambda i: (i, 0)),
            pl.BlockSpec((BN, F), lambda i: (i, 0)),
            pl.BlockSpec((BN, 1), lambda i: (i, 0)),
        ],
        out_specs=[
            pl.BlockSpec((BN, F), lambda i: (i, 0)),
            pl.BlockSpec((BN, F), lambda i: (i, 0)),
        ],
        out_shape=[
            jax.ShapeDtypeStruct((n, F), jnp.float32),
            jax.ShapeDtypeStruct((n, F), jnp.float32),
        ],
    )(alo, ahi, s)


def _scale_add_body(alo_ref, ahi_ref, s_ref, blo_ref, bhi_ref, olo_ref,
                    ohi_ref):
    sb = s_ref[...]
    olo_ref[...] = alo_ref[...] * sb + blo_ref[...]
    ohi_ref[...] = ahi_ref[...] * sb + bhi_ref[...]


def _tc_scale_add(alo, ahi, s, blo, bhi):
    n = alo.shape[0]
    g = n // BN
    return pl.pallas_call(
        _scale_add_body,
        grid=(g,),
        in_specs=[pl.BlockSpec((BN, F), lambda i: (i, 0)),
                  pl.BlockSpec((BN, F), lambda i: (i, 0)),
                  pl.BlockSpec((BN, 1), lambda i: (i, 0)),
                  pl.BlockSpec((BN, F), lambda i: (i, 0)),
                  pl.BlockSpec((BN, F), lambda i: (i, 0))],
        out_specs=[pl.BlockSpec((BN, F), lambda i: (i, 0)),
                   pl.BlockSpec((BN, F), lambda i: (i, 0))],
        out_shape=[jax.ShapeDtypeStruct((n, F), jnp.float32),
                   jax.ShapeDtypeStruct((n, F), jnp.float32)],
    )(alo, ahi, s, blo, bhi)


def _mid_body(x_ref, p1lo_ref, p1hi_ref, p2lo_ref, p2hi_ref, n_ref, W1_ref,
              b1_ref, W2r_ref, v1_ref, v2nlo_ref, v2nhi_ref, t2lo_ref,
              t2hi_ref):
    nb = n_ref[...]
    cat = jnp.concatenate(
        [x_ref[...],
         p1lo_ref[...] * nb, p1hi_ref[...] * nb,
         p2lo_ref[...] * nb, p2hi_ref[...] * nb], axis=1)
    h1 = jnp.dot(cat, W1_ref[...], preferred_element_type=jnp.float32)
    h1 = jnp.maximum(h1 + b1_ref[...], 0.0)
    v = jnp.dot(h1, W2r_ref[...], preferred_element_type=jnp.float32)
    v1_ref[...] = v[:, :2 * F]
    v2nlo_ref[...] = v[:, 2 * F:3 * F] * nb
    v2nhi_ref[...] = v[:, 3 * F:4 * F] * nb
    t2lo_ref[...] = v[:, 4 * F:5 * F] * nb
    t2hi_ref[...] = v[:, 5 * F:6 * F] * nb


def _tc_mid(x, p1lo, p1hi, p2lo, p2hi, norm, W1, b1, W2r):
    n = x.shape[0]
    g = n // BN
    in_dim = x.shape[1]
    hid = W1.shape[1]
    return pl.pallas_call(
        _mid_body,
        grid=(g,),
        in_specs=[
            pl.BlockSpec((BN, in_dim), lambda i: (i, 0)),
            pl.BlockSpec((BN, F), lambda i: (i, 0)),
            pl.BlockSpec((BN, F), lambda i: (i, 0)),
            pl.BlockSpec((BN, F), lambda i: (i, 0)),
            pl.BlockSpec((BN, F), lambda i: (i, 0)),
            pl.BlockSpec((BN, 1), lambda i: (i, 0)),
            pl.BlockSpec(W1.shape, lambda i: (0, 0)),
            pl.BlockSpec((1, hid), lambda i: (0, 0)),
            pl.BlockSpec(W2r.shape, lambda i: (0, 0)),
        ],
        out_specs=[
            pl.BlockSpec((BN, 2 * F), lambda i: (i, 0)),
            pl.BlockSpec((BN, F), lambda i: (i, 0)),
            pl.BlockSpec((BN, F), lambda i: (i, 0)),
            pl.BlockSpec((BN, F), lambda i: (i, 0)),
            pl.BlockSpec((BN, F), lambda i: (i, 0)),
        ],
        out_shape=[
            jax.ShapeDtypeStruct((n, 2 * F), jnp.float32),
            jax.ShapeDtypeStruct((n, F), jnp.float32),
            jax.ShapeDtypeStruct((n, F), jnp.float32),
            jax.ShapeDtypeStruct((n, F), jnp.float32),
            jax.ShapeDtypeStruct((n, F), jnp.float32),
        ],
    )(x, p1lo, p1hi, p2lo, p2hi, norm, W1, b1, W2r)


def _head_body(g, n, v1_ref, qlo_ref, qhi_ref, n_ref, b2_ref, Wc_ref, bc_ref,
               y_ref, acc_ref):
    i = pl.program_id(0)
    nb = n_ref[...]
    h2 = jnp.concatenate([qlo_ref[...], qhi_ref[...]], axis=1) * nb
    h2 = jnp.maximum(h2 + v1_ref[...] + b2_ref[...], 0.0)
    part = jnp.sum(h2, axis=0, keepdims=True)

    @pl.when(i == 0)
    def _():
        acc_ref[...] = part

    @pl.when(i > 0)
    def _():
        acc_ref[...] = acc_ref[...] + part

    @pl.when(i == g - 1)
    def _():
        hg = acc_ref[...] * (1.0 / n)
        y_ref[...] = (jnp.dot(hg, Wc_ref[...],
                              preferred_element_type=jnp.float32)
                      + bc_ref[...])


def _tc_head(v1, qlo, qhi, norm, b2, Wc, bc):
    n = v1.shape[0]
    g = n // BN
    ncls = Wc.shape[1]
    return pl.pallas_call(
        functools.partial(_head_body, g, float(n)),
        grid=(g,),
        in_specs=[
            pl.BlockSpec((BN, 2 * F), lambda i: (i, 0)),
            pl.BlockSpec((BN, F), lambda i: (i, 0)),
            pl.BlockSpec((BN, F), lambda i: (i, 0)),
            pl.BlockSpec((BN, 1), lambda i: (i, 0)),
            pl.BlockSpec((1, 2 * F), lambda i: (0, 0)),
            pl.BlockSpec(Wc.shape, lambda i: (0, 0)),
            pl.BlockSpec((1, ncls), lambda i: (0, 0)),
        ],
        out_specs=pl.BlockSpec((1, ncls), lambda i: (0, 0)),
        out_shape=jax.ShapeDtypeStruct((1, ncls), jnp.float32),
        scratch_shapes=[pltpu.VMEM((1, 2 * F), jnp.float32)],
    )(v1, qlo, qhi, norm, b2, Wc, bc)


def kernel(x, edge_index, W1, b1, W2, b2, Wc, bc):
    n, in_dim = x.shape
    e = edge_index.shape[1]
    hid = W1.shape[1]
    out2 = W2.shape[1]
    assert in_dim == 2 * F and n % BN == 0 and n % NS == 0
    assert e % (NC * NS) == 0 and (e // NS) % _CH == 0 and (e // (NC * NS)) % 8 == 0

    src = edge_index[0]
    dst = edge_index[1]
    n_pad = ((n + NS * LANES - 1) // (NS * LANES)) * NS * LANES
    zeros = jnp.zeros((n_pad, F), jnp.float32)

    W2r = jnp.concatenate([W2[:hid], W2[hid:2 * hid], W2[2 * hid:]], axis=1)

    deg2 = _sc_degree(dst, jnp.ones((_CHD, F), jnp.float32),
                      zeros, n_pad, e)
    degcol = (deg2[:n, 0] + deg2[n_pad:n_pad + n, 0]).reshape(n, 1)
    t0lo, t0hi, norm, norm2 = _tc_prep(degcol, x)
    p1lo, p1hi = _sc_prop(t0lo, t0hi, src, dst, zeros, n_pad)
    t1lo, t1hi = _tc_scale(p1lo, p1hi, norm2)
    p2lo, p2hi = _sc_prop(t1lo, t1hi, src, dst, zeros, n_pad)
    v1, v2nlo, v2nhi, t2lo, t2hi = _tc_mid(
        x, p1lo, p1hi, p2lo, p2hi, norm, W1, b1.reshape(1, hid), W2r)
    q1lo, q1hi = _sc_prop(t2lo, t2hi, src, dst, zeros, n_pad)
    t3lo, t3hi = _tc_scale_add(q1lo, q1hi, norm2, v2nlo, v2nhi)
    q2lo, q2hi = _sc_prop(t3lo, t3hi, src, dst, zeros, n_pad)
    y = _tc_head(v1, q2lo, q2hi, norm, b2.reshape(1, out2), Wc,
                 bc.reshape(1, -1))
    return y


# speedup vs baseline: 8.9094x; 2.0071x over previous
import functools

import jax
import jax.numpy as jnp
from jax import lax
from jax.experimental import pallas as pl
from jax.experimental.pallas import tpu as pltpu
from jax.experimental.pallas import tpu_sc as plsc

NC = 2
NS = 16
LANES = 16
F = 128
BN = 400


def _sc_mesh():
    return plsc.VectorSubcoreMesh(core_axis_name="c", subcore_axis_name="s")


_CH = 128


def _degree_body(n_pad, nchd, dst_hbm, ones_hbm, z_hbm, out_hbm, didx, onesv,
                 acc, sem0, sem1):
    c = lax.axis_index("c")
    s = lax.axis_index("s")
    wid = c * NS + s
    npw = n_pad // NS
    sems = (sem0, sem1)

    pltpu.sync_copy(ones_hbm, onesv)
    pltpu.sync_copy(dst_hbm.at[wid], didx)
    pltpu.sync_copy(z_hbm.at[pl.ds(s * npw, npw)], acc.at[pl.ds(s * npw, npw)])
    plsc.subcore_barrier()

    def gbody(g, carry):
        for b in range(2):
            k = g * 2 + b

            @pl.when(k >= 2)
            def _():
                pltpu.make_async_copy(onesv, acc.at[didx.at[k - 2]],
                                      sems[b]).wait()

            pltpu.async_copy(onesv, acc.at[didx.at[k]], sems[b], add=True)
        return carry

    lax.fori_loop(0, nchd // 2, gbody, 0)
    for b in range(2):
        pltpu.make_async_copy(onesv, acc.at[didx.at[nchd - 2 + b]],
                              sems[b]).wait()

    plsc.subcore_barrier()
    pltpu.sync_copy(acc.at[pl.ds(s * npw, npw)],
                    out_hbm.at[pl.ds(c * n_pad + s * npw, npw)])


def _sc_degree(dst3d, ones, zeros2d, n_pad):
    nchd = dst3d.shape[1]
    kfn = pl.kernel(
        functools.partial(_degree_body, n_pad, nchd),
        out_type=jax.ShapeDtypeStruct((NC * n_pad, F), jnp.float32),
        mesh=_sc_mesh(),
        scratch_types=[
            pltpu.VMEM((nchd, _CH), jnp.int32),
            pltpu.VMEM((_CH, F), jnp.float32),
            pltpu.VMEM_SHARED((n_pad, F), jnp.float32),
            pltpu.SemaphoreType.DMA,
            pltpu.SemaphoreType.DMA,
        ],
    )
    return kfn(dst3d, ones, zeros2d)


def _prop_body(n_pad, nch, tlo_hbm, thi_hbm, src_hbm, dst_hbm, z_hbm, olo_hbm,
               ohi_hbm, didx, sb0, sb1, rows0, rows1, acc,
               is0, is1, gs0, gs1, ss0, ss1):
    c = lax.axis_index("c")
    s = lax.axis_index("s")
    rpt = n_pad // NS
    sbuf = (sb0, sb1)
    rows = (rows0, rows1)
    isem = (is0, is1)
    gsem = (gs0, gs1)
    ssem = (ss0, ss1)

    pltpu.sync_copy(dst_hbm.at[s], didx)
    pltpu.sync_copy(z_hbm.at[pl.ds(s * rpt, rpt)], acc.at[pl.ds(s * rpt, rpt)])
    plsc.subcore_barrier()

    def edge_loop(table):
        pltpu.sync_copy(src_hbm.at[s, 0], sb0)
        pltpu.async_copy(table.at[sb0.at[0]], rows0, gs0)
        pltpu.async_copy(src_hbm.at[s, 1], sb1, is1)

        def gbody(g, carry):
            for b in range(2):
                k = g * 2 + b
                o = 1 - b
                nk = k + 1

                @pl.when(nk < nch)
                def _():
                    @pl.when(k >= 1)
                    def _():
                        pltpu.make_async_copy(
                            rows[o], acc.at[didx.at[k - 1]], ssem[o]).wait()

                    pltpu.make_async_copy(src_hbm.at[s, nk], sbuf[o],
                                          isem[o]).wait()
                    pltpu.async_copy(table.at[sbuf[o].at[0]], rows[o],
                                     gsem[o])

                pltpu.make_async_copy(table.at[sbuf[b].at[0]], rows[b],
                                      gsem[b]).wait()

                @pl.when(nk + 1 < nch)
                def _():
                    pltpu.async_copy(src_hbm.at[s, nk + 1], sbuf[b], isem[b])

                pltpu.async_copy(rows[b], acc.at[didx.at[k]], ssem[b],
                                 add=True)
            return carry

        lax.fori_loop(0, nch // 2, gbody, 0)
        for b in range(2):
            pltpu.make_async_copy(rows[b], acc.at[didx.at[nch - 2 + b]],
                                  ssem[b]).wait()

    @pl.when(c == 0)
    def _():
        edge_loop(tlo_hbm)

    @pl.when(c == 1)
    def _():
        edge_loop(thi_hbm)

    plsc.subcore_barrier()

    @pl.when(c == 0)
    def _():
        pltpu.sync_copy(acc.at[pl.ds(s * rpt, rpt)],
                        olo_hbm.at[pl.ds(s * rpt, rpt)])

    @pl.when(c == 1)
    def _():
        pltpu.sync_copy(acc.at[pl.ds(s * rpt, rpt)],
                        ohi_hbm.at[pl.ds(s * rpt, rpt)])


def _sc_prop(tlo, thi, src4, dst3, zeros, n_pad):
    nch = src4.shape[1]
    kfn = pl.kernel(
        functools.partial(_prop_body, n_pad, nch),
        out_type=(jax.ShapeDtypeStruct((n_pad, F), jnp.float32),
                  jax.ShapeDtypeStruct((n_pad, F), jnp.float32)),
        mesh=_sc_mesh(),
        scratch_types=[
            pltpu.VMEM((nch, _CH), jnp.int32),
            pltpu.VMEM((1, _CH), jnp.int32),
            pltpu.VMEM((1, _CH), jnp.int32),
            pltpu.VMEM((_CH, F), jnp.float32),
            pltpu.VMEM((_CH, F), jnp.float32),
            pltpu.VMEM_SHARED((n_pad, F), jnp.float32),
        ] + [pltpu.SemaphoreType.DMA] * 6,
    )
    return kfn(tlo, thi, src4, dst3, zeros)


def _prep_body(deg_ref, x_ref, t0lo_ref, t0hi_ref, norm_ref, norm2_ref):
    d = jnp.maximum(deg_ref[...], 1.0)
    norm = lax.rsqrt(d)
    xb = x_ref[...]
    t0lo_ref[...] = xb[:, :F] * norm
    t0hi_ref[...] = xb[:, F:] * norm
    norm_ref[...] = norm
    norm2_ref[...] = 1.0 / d


def _tc_prep(deg2, x):
    n = x.shape[0]
    g = n // BN
    return pl.pallas_call(
        _prep_body,
        grid=(g,),
        in_specs=[
            pl.BlockSpec((BN, 1), lambda i: (i, 0)),
            pl.BlockSpec((BN, 2 * F), lambda i: (i, 0)),
        ],
        out_specs=[
            pl.BlockSpec((BN, F), lambda i: (i, 0)),
            pl.BlockSpec((BN, F), lambda i: (i, 0)),
            pl.BlockSpec((BN, 1), lambda i: (i, 0)),
            pl.BlockSpec((BN, 1), lambda i: (i, 0)),
        ],
        out_shape=[
            jax.ShapeDtypeStruct((n, F), jnp.float32),
            jax.ShapeDtypeStruct((n, F), jnp.float32),
            jax.ShapeDtypeStruct((n, 1), jnp.float32),
            jax.ShapeDtypeStruct((n, 1), jnp.float32),
        ],
    )(deg2, x)


def _scale_body(alo_ref, ahi_ref, s_ref, olo_ref, ohi_ref):
    sb = s_ref[...]
    olo_ref[...] = alo_ref[...] * sb
    ohi_ref[...] = ahi_ref[...] * sb


def _tc_scale(alo, ahi, s):
    n = alo.shape[0]
    g = n // BN
    return pl.pallas_call(
        _scale_body,
        grid=(g,),
        in_specs=[
            pl.BlockSpec((BN, F), lambda i: (i, 0)),
            pl.BlockSpec((BN, F), lambda i: (i, 0)),
            pl.BlockSpec((BN, 1), lambda i: (i, 0)),
        ],
        out_specs=[
            pl.BlockSpec((BN, F), lambda i: (i, 0)),
            pl.BlockSpec((BN, F), lambda i: (i, 0)),
        ],
        out_shape=[
            jax.ShapeDtypeStruct((n, F), jnp.float32),
            jax.ShapeDtypeStruct((n, F), jnp.float32),
        ],
    )(alo, ahi, s)


def _scale_add_body(alo_ref, ahi_ref, s_ref, blo_ref, bhi_ref, olo_ref,
                    ohi_ref):
    sb = s_ref[...]
    olo_ref[...] = alo_ref[...] * sb + blo_ref[...]
    ohi_ref[...] = ahi_ref[...] * sb + bhi_ref[...]


def _tc_scale_add(alo, ahi, s, blo, bhi):
    n = alo.shape[0]
    g = n // BN
    return pl.pallas_call(
        _scale_add_body,
        grid=(g,),
        in_specs=[pl.BlockSpec((BN, F), lambda i: (i, 0)),
                  pl.BlockSpec((BN, F), lambda i: (i, 0)),
                  pl.BlockSpec((BN, 1), lambda i: (i, 0)),
                  pl.BlockSpec((BN, F), lambda i: (i, 0)),
                  pl.BlockSpec((BN, F), lambda i: (i, 0))],
        out_specs=[pl.BlockSpec((BN, F), lambda i: (i, 0)),
                   pl.BlockSpec((BN, F), lambda i: (i, 0))],
        out_shape=[jax.ShapeDtypeStruct((n, F), jnp.float32),
                   jax.ShapeDtypeStruct((n, F), jnp.float32)],
    )(alo, ahi, s, blo, bhi)


def _mid_body(x_ref, p1lo_ref, p1hi_ref, p2lo_ref, p2hi_ref, n_ref, W1_ref,
              b1_ref, W2r_ref, v1_ref, v2nlo_ref, v2nhi_ref, t2lo_ref,
              t2hi_ref):
    nb = n_ref[...]
    cat = jnp.concatenate(
        [x_ref[...],
         p1lo_ref[...] * nb, p1hi_ref[...] * nb,
         p2lo_ref[...] * nb, p2hi_ref[...] * nb], axis=1)
    h1 = jnp.dot(cat, W1_ref[...], preferred_element_type=jnp.float32)
    h1 = jnp.maximum(h1 + b1_ref[...], 0.0)
    v = jnp.dot(h1, W2r_ref[...], preferred_element_type=jnp.float32)
    v1_ref[...] = v[:, :2 * F]
    v2nlo_ref[...] = v[:, 2 * F:3 * F] * nb
    v2nhi_ref[...] = v[:, 3 * F:4 * F] * nb
    t2lo_ref[...] = v[:, 4 * F:5 * F] * nb
    t2hi_ref[...] = v[:, 5 * F:6 * F] * nb


def _tc_mid(x, p1lo, p1hi, p2lo, p2hi, norm, W1, b1, W2r):
    n = x.shape[0]
    g = n // BN
    in_dim = x.shape[1]
    hid = W1.shape[1]
    return pl.pallas_call(
        _mid_body,
        grid=(g,),
        in_specs=[
            pl.BlockSpec((BN, in_dim), lambda i: (i, 0)),
            pl.BlockSpec((BN, F), lambda i: (i, 0)),
            pl.BlockSpec((BN, F), lambda i: (i, 0)),
            pl.BlockSpec((BN, F), lambda i: (i, 0)),
            pl.BlockSpec((BN, F), lambda i: (i, 0)),
            pl.BlockSpec((BN, 1), lambda i: (i, 0)),
            pl.BlockSpec(W1.shape, lambda i: (0, 0)),
            pl.BlockSpec((1, hid), lambda i: (0, 0)),
            pl.BlockSpec(W2r.shape, lambda i: (0, 0)),
        ],
        out_specs=[
            pl.BlockSpec((BN, 2 * F), lambda i: (i, 0)),
            pl.BlockSpec((BN, F), lambda i: (i, 0)),
            pl.BlockSpec((BN, F), lambda i: (i, 0)),
            pl.BlockSpec((BN, F), lambda i: (i, 0)),
            pl.BlockSpec((BN, F), lambda i: (i, 0)),
        ],
        out_shape=[
            jax.ShapeDtypeStruct((n, 2 * F), jnp.float32),
            jax.ShapeDtypeStruct((n, F), jnp.float32),
            jax.ShapeDtypeStruct((n, F), jnp.float32),
            jax.ShapeDtypeStruct((n, F), jnp.float32),
            jax.ShapeDtypeStruct((n, F), jnp.float32),
        ],
    )(x, p1lo, p1hi, p2lo, p2hi, norm, W1, b1, W2r)


def _head_body(g, n, v1_ref, qlo_ref, qhi_ref, n_ref, b2_ref, Wc_ref, bc_ref,
               y_ref, acc_ref):
    i = pl.program_id(0)
    nb = n_ref[...]
    h2 = jnp.concatenate([qlo_ref[...], qhi_ref[...]], axis=1) * nb
    h2 = jnp.maximum(h2 + v1_ref[...] + b2_ref[...], 0.0)
    part = jnp.sum(h2, axis=0, keepdims=True)

    @pl.when(i == 0)
    def _():
        acc_ref[...] = part

    @pl.when(i > 0)
    def _():
        acc_ref[...] = acc_ref[...] + part

    @pl.when(i == g - 1)
    def _():
        hg = acc_ref[...] * (1.0 / n)
        y_ref[...] = (jnp.dot(hg, Wc_ref[...],
                              preferred_element_type=jnp.float32)
                      + bc_ref[...])


def _tc_head(v1, qlo, qhi, norm, b2, Wc, bc):
    n = v1.shape[0]
    g = n // BN
    ncls = Wc.shape[1]
    return pl.pallas_call(
        functools.partial(_head_body, g, float(n)),
        grid=(g,),
        in_specs=[
            pl.BlockSpec((BN, 2 * F), lambda i: (i, 0)),
            pl.BlockSpec((BN, F), lambda i: (i, 0)),
            pl.BlockSpec((BN, F), lambda i: (i, 0)),
            pl.BlockSpec((BN, 1), lambda i: (i, 0)),
            pl.BlockSpec((1, 2 * F), lambda i: (0, 0)),
            pl.BlockSpec(Wc.shape, lambda i: (0, 0)),
            pl.BlockSpec((1, ncls), lambda i: (0, 0)),
        ],
        out_specs=pl.BlockSpec((1, ncls), lambda i: (0, 0)),
        out_shape=jax.ShapeDtypeStruct((1, ncls), jnp.float32),
        scratch_shapes=[pltpu.VMEM((1, 2 * F), jnp.float32)],
    )(v1, qlo, qhi, norm, b2, Wc, bc)


def kernel(x, edge_index, W1, b1, W2, b2, Wc, bc):
    n, in_dim = x.shape
    e = edge_index.shape[1]
    hid = W1.shape[1]
    out2 = W2.shape[1]
    assert in_dim == 2 * F and n % BN == 0 and n % NS == 0

    src = edge_index[0]
    dst = edge_index[1]
    n_pad = ((n + NS * LANES - 1) // (NS * LANES)) * NS * LANES
    assert n_pad > n
    zeros = jnp.zeros((n_pad, F), jnp.float32)

    nch = -(-e // (NS * _CH))
    nch = ((nch + 3) // 4) * 4
    ep = NS * nch * _CH
    nchd = nch // NC
    ar = jnp.arange(ep - e, dtype=jnp.int32)
    src_p = jnp.concatenate([src, ar % n])
    dst_p = jnp.concatenate([dst, n + ar % (n_pad - n)])
    src4 = src_p.reshape(NS, nch, 1, _CH)
    dst3 = dst_p.reshape(NS, nch, _CH)
    dst3d = dst_p.reshape(NC * NS, nchd, _CH)

    W2r = jnp.concatenate([W2[:hid], W2[hid:2 * hid], W2[2 * hid:]], axis=1)

    deg2 = _sc_degree(dst3d, jnp.ones((_CH, F), jnp.float32), zeros, n_pad)
    degcol = (deg2[:n, 0] + deg2[n_pad:n_pad + n, 0]).reshape(n, 1)
    t0lo, t0hi, norm, norm2 = _tc_prep(degcol, x)
    p1lo, p1hi = _sc_prop(t0lo, t0hi, src4, dst3, zeros, n_pad)
    t1lo, t1hi = _tc_scale(p1lo, p1hi, norm2)
    p2lo, p2hi = _sc_prop(t1lo, t1hi, src4, dst3, zeros, n_pad)
    v1, v2nlo, v2nhi, t2lo, t2hi = _tc_mid(
        x, p1lo, p1hi, p2lo, p2hi, norm, W1, b1.reshape(1, hid), W2r)
    q1lo, q1hi = _sc_prop(t2lo, t2hi, src4, dst3, zeros, n_pad)
    t3lo, t3hi = _tc_scale_add(q1lo, q1hi, norm2, v2nlo, v2nhi)
    q2lo, q2hi = _sc_prop(t3lo, t3hi, src4, dst3, zeros, n_pad)
    y = _tc_head(v1, q2lo, q2hi, norm, b2.reshape(1, out2), Wc,
                 bc.reshape(1, -1))
    return y
